# baseline (speedup 1.0000x reference)
import functools

import jax
import jax.numpy as jnp
from jax import lax
from jax.experimental import pallas as pl
from jax.experimental.pallas import tpu as pltpu
from jax.experimental.pallas import tpu_sc as plsc

B = 16384
F = 32
L = 16
NC = 2
NS = 16
NW = NC * NS
BPW = B // NW
NCHUNK = 4
CHUNK = BPW // NCHUNK
NGROUP = BPW // L

_mesh = plsc.VectorSubcoreMesh(core_axis_name="c", subcore_axis_name="s")


@functools.partial(
    pl.kernel,
    out_type=jax.ShapeDtypeStruct((B,), jnp.float32),
    mesh=_mesh,
    compiler_params=pltpu.CompilerParams(
        needs_layout_passes=False, use_tc_tiling_on_sc=False),
    scratch_types=[
        pltpu.VMEM((NCHUNK, CHUNK), jnp.int32),
        pltpu.VMEM((NCHUNK, CHUNK), jnp.int32),
        pltpu.VMEM((BPW, F), jnp.float32),
        pltpu.VMEM((BPW, F), jnp.float32),
        pltpu.VMEM((2, BPW), jnp.float32),
        pltpu.VMEM((F * L + 3 * L,), jnp.float32),
        pltpu.VMEM((BPW,), jnp.float32),
        pltpu.SemaphoreType.DMA,
    ],
)
def _sc_fused(user_hbm, item_hbm, ctx_hbm, params_hbm, utab_hbm, itab_hbm,
              out_hbm, uidx, iidx, ubuf, vbuf, ctxbuf, pbuf, obuf, sem):
    wid = lax.axis_index("s") * NC + lax.axis_index("c")

    pltpu.sync_copy(user_hbm.at[wid], uidx)
    pltpu.sync_copy(item_hbm.at[wid], iidx)
    pltpu.sync_copy(ctx_hbm.at[wid], ctxbuf)
    pltpu.sync_copy(params_hbm, pbuf)

    copies = []
    for j in range(NCHUNK):
        dst = ubuf.at[pl.ds(j * CHUNK, CHUNK), :]
        copies.append(pltpu.async_copy(utab_hbm.at[uidx.at[j]], dst, sem))
    for j in range(NCHUNK):
        dst = vbuf.at[pl.ds(j * CHUNK, CHUNK), :]
        copies.append(pltpu.async_copy(itab_hbm.at[iidx.at[j]], dst, sem))
    for c in copies:
        c.wait()

    g0 = pbuf[F * L:F * L + L]
    g1 = pbuf[F * L + L:F * L + 2 * L]
    base = pbuf[F * L + 2 * L:F * L + 3 * L]

    lanes = lax.iota(jnp.int32, L)

    def g_body(g, carry):
        off = g * L
        rows = off + lanes
        c0 = ctxbuf[0, pl.ds(off, L)]
        c1 = ctxbuf[1, pl.ds(off, L)]
        acc = base + c0 * g0 + c1 * g1
        for f in range(F):
            fv = jnp.full((L,), f, jnp.int32)
            w = pbuf[f * L:(f + 1) * L]
            ug = plsc.load_gather(ubuf, [rows, fv])
            vg = plsc.load_gather(vbuf, [rows, fv])
            acc = acc + ug * vg * w
        obuf[pl.ds(off, L)] = acc
        return carry

    lax.fori_loop(0, NGROUP, g_body, 0)

    pltpu.sync_copy(obuf, out_hbm.at[pl.ds(wid * BPW, BPW)])


def kernel(user, item, context, user_table, item_table, Wc, bc, Wo, bo):
    user3 = user.astype(jnp.int32).reshape(NW, NCHUNK, CHUNK)
    item3 = item.astype(jnp.int32).reshape(NW, NCHUNK, CHUNK)
    ctx3 = context.reshape(NW, BPW, 2).transpose(0, 2, 1)
    wo = Wo.reshape(F)
    g = Wc @ wo
    base = bc @ wo + bo[0]
    params = jnp.concatenate(
        [jnp.broadcast_to(wo[:, None], (F, L)).reshape(F * L),
         jnp.full((L,), g[0], jnp.float32),
         jnp.full((L,), g[1], jnp.float32),
         jnp.full((L,), base, jnp.float32)]
    )
    return _sc_fused(user3, item3, ctx3, params, user_table, item_table)
